# baseline (speedup 1.0000x reference)
import functools

import jax
import jax.numpy as jnp
from jax.experimental import pallas as pl
from jax.experimental.pallas import tpu as pltpu

_N = 4096
_D = 64
_K = 4
_CH = 256
_NSTRIPE = 4
_NSTEP = _N // (_CH * _NSTRIPE)


def _stripe_index(i, s):
    return (_NSTRIPE * i + s, 0)


def _mixed_op_body(m0, m1, m2, m3, h_ref, w_ref, out_ref, hc_ref):
    @pl.when(pl.program_id(0) == 0)
    def _combine():
        acc = jnp.zeros((_N, _D), jnp.float32)
        for k in range(_K):
            wk = w_ref[k]
            acc = acc + jnp.where(wk > 0, wk, 0.0) * h_ref[k]
        hc_ref[...] = acc.astype(jnp.bfloat16)

    c = jnp.float32(0.0)
    for k in range(_K):
        wk = w_ref[k]
        c = c + jnp.where(wk > 0, jnp.float32(0.0), wk)
    hc = hc_ref[...]
    for s, m in enumerate((m0, m1, m2, m3)):
        out_ref[pl.ds(s * _CH, _CH), :] = (
            jnp.dot(
                m[...].astype(jnp.bfloat16),
                hc,
                preferred_element_type=jnp.float32,
            )
            + c
        )


@jax.jit
def kernel(mask_matrix, h_op_list, weights):
    stripe_specs = [
        pl.BlockSpec((_CH, _N), functools.partial(_stripe_index, s=s))
        for s in range(_NSTRIPE)
    ]
    return pl.pallas_call(
        _mixed_op_body,
        grid=(_NSTEP,),
        in_specs=stripe_specs
        + [
            pl.BlockSpec((_K, _N, _D), lambda i: (0, 0, 0)),
            pl.BlockSpec(memory_space=pltpu.SMEM),
        ],
        out_specs=pl.BlockSpec((_CH * _NSTRIPE, _D), lambda i: (i, 0)),
        out_shape=jax.ShapeDtypeStruct((_N, _D), jnp.float32),
        scratch_shapes=[
            pltpu.VMEM((_N, _D), jnp.bfloat16),
        ],
    )(mask_matrix, mask_matrix, mask_matrix, mask_matrix, h_op_list, weights)


# speedup vs baseline: 1.0129x; 1.0129x over previous
import jax
import jax.numpy as jnp
from jax.experimental import pallas as pl
from jax.experimental.pallas import tpu as pltpu

_N = 4096
_D = 64
_K = 4
_BN = 256
_NBUF = 6


def _mixed_op_body(mask_hbm, h_ref, w_ref, out_hbm, hc_ref):
    acc = jnp.zeros((_N, _D), jnp.float32)
    c = jnp.float32(0.0)
    for k in range(_K):
        wk = w_ref[k]
        acc = acc + jnp.where(wk > 0, wk, 0.0) * h_ref[k]
        c = c + jnp.where(wk > 0, jnp.float32(0.0), wk)
    hc_ref[...] = acc.astype(jnp.bfloat16)

    def _inner(mask_blk, out_blk):
        out_blk[...] = (
            jnp.dot(
                mask_blk[...].astype(jnp.bfloat16),
                hc_ref[...],
                preferred_element_type=jnp.float32,
            )
            + c
        )

    pltpu.emit_pipeline(
        _inner,
        grid=(_N // _BN,),
        in_specs=[
            pl.BlockSpec(
                (_BN, _N),
                lambda i: (i, 0),
                pipeline_mode=pl.Buffered(buffer_count=_NBUF),
            )
        ],
        out_specs=[pl.BlockSpec((_BN, _D), lambda i: (i, 0))],
    )(mask_hbm, out_hbm)


@jax.jit
def kernel(mask_matrix, h_op_list, weights):
    return pl.pallas_call(
        _mixed_op_body,
        in_specs=[
            pl.BlockSpec(memory_space=pltpu.HBM),
            pl.BlockSpec((_K, _N, _D), lambda: (0, 0, 0)),
            pl.BlockSpec(memory_space=pltpu.SMEM),
        ],
        out_specs=pl.BlockSpec(memory_space=pltpu.HBM),
        out_shape=jax.ShapeDtypeStruct((_N, _D), jnp.float32),
        scratch_shapes=[
            pltpu.VMEM((_N, _D), jnp.bfloat16),
        ],
    )(mask_matrix, h_op_list, weights)
